# baseline (speedup 1.0000x reference)
import functools

import jax
import jax.numpy as jnp
from jax import lax
from jax.experimental import pallas as pl
from jax.experimental.pallas import tpu as pltpu
from jax.experimental.pallas import tpu_sc as plsc


def _build_sc_kernel(B, L, E, n_workers, num_cores):
    per_w = B // n_workers
    n_pair = per_w // 2

    mesh = plsc.VectorSubcoreMesh(core_axis_name="c", subcore_axis_name="s")

    @functools.partial(
        pl.kernel,
        out_type=jax.ShapeDtypeStruct((L, B, 128), jnp.float32),
        mesh=mesh,
        scratch_types=[
            pltpu.VMEM((per_w, L), jnp.int32),
            pltpu.VMEM((L, E), jnp.float32),
            pltpu.VMEM((L, E), jnp.float32),
            pltpu.SemaphoreType.DMA,
            pltpu.SemaphoreType.DMA,
            pltpu.SemaphoreType.DMA,
            pltpu.SemaphoreType.DMA,
        ],
        compiler_params=pltpu.CompilerParams(use_tc_tiling_on_sc=False),
    )
    def emb(ids_hbm, wt_hbm, out_hbm, ids_v, rows0, rows1, g0, g1, s0, s1):
        cid = lax.axis_index("c")
        sid = lax.axis_index("s")
        wid = sid * num_cores + cid
        base = wid * per_w

        pltpu.sync_copy(ids_hbm.at[pl.ds(base, per_w)], ids_v)

        def gather(i, rows, gsem):
            return pltpu.make_async_copy(wt_hbm.at[ids_v.at[i]], rows, gsem)

        def store(i, rows, ssem):
            return pltpu.make_async_copy(
                rows, out_hbm.at[:, base + i, pl.ds(0, E)], ssem)

        def step(i, rows_a, rows_b, gsem_a, gsem_b, ssem_a, ssem_b):
            gather(i, rows_a, gsem_a).wait()

            @pl.when(i > 0)
            def _():
                store(i - 1, rows_b, ssem_b).wait()

            @pl.when(i + 1 < per_w)
            def _():
                gather(i + 1, rows_b, gsem_b).start()

            store(i, rows_a, ssem_a).start()

        gather(0, rows0, g0).start()

        def pair_body(k, carry):
            step(2 * k, rows0, rows1, g0, g1, s0, s1)
            step(2 * k + 1, rows1, rows0, g1, g0, s1, s0)
            return carry

        lax.fori_loop(0, n_pair, pair_body, 0)
        store(per_w - 1, rows1, s1).wait()

    return emb


def _build_tc_kernel(B, L, E):
    BB = 512

    def body(in_ref, pos_ref, out_ref):
        l = pl.program_id(0)
        x = in_ref[0][:, :E]
        p = pos_ref[l, :]
        out_ref[0] = x.T + p[:, None]

    return pl.pallas_call(
        body,
        grid=(L, B // BB),
        in_specs=[
            pl.BlockSpec((1, BB, 128), lambda l, bb: (l, bb, 0)),
            pl.BlockSpec((512, E), lambda l, bb: (0, 0)),
        ],
        out_specs=pl.BlockSpec((1, E, BB), lambda l, bb: (l, 0, bb)),
        out_shape=jax.ShapeDtypeStruct((L, E, B), jnp.float32),
    )


def kernel(input_ids, word_table, pos_table):
    B, L = input_ids.shape
    E = word_table.shape[1]
    info = plsc.get_sparse_core_info()
    n_workers = info.num_cores * info.num_subcores

    emb = _build_sc_kernel(B, L, E, n_workers, info.num_cores)
    inter = emb(input_ids.astype(jnp.int32), word_table)
    out_leb = _build_tc_kernel(B, L, E)(inter, pos_table)
    return jnp.transpose(out_leb, (2, 0, 1))


# speedup vs baseline: 2.3623x; 2.3623x over previous
import functools

import jax
import jax.numpy as jnp
from jax import lax
from jax.experimental import pallas as pl
from jax.experimental.pallas import tpu as pltpu
from jax.experimental.pallas import tpu_sc as plsc


def _build_sc_kernel(B, L, E, n_workers, num_cores):
    per_w = B // n_workers
    n_pair = per_w // 2

    mesh = plsc.VectorSubcoreMesh(core_axis_name="c", subcore_axis_name="s")

    @functools.partial(
        pl.kernel,
        out_type=jax.ShapeDtypeStruct((L, B, 128), jnp.float32),
        mesh=mesh,
        scratch_types=[
            pltpu.VMEM((per_w, L), jnp.int32),
            pltpu.VMEM((L, E), jnp.float32),
            pltpu.VMEM((L, E), jnp.float32),
            pltpu.SemaphoreType.DMA,
            pltpu.SemaphoreType.DMA,
            pltpu.SemaphoreType.DMA,
            pltpu.SemaphoreType.DMA,
        ],
        compiler_params=pltpu.CompilerParams(use_tc_tiling_on_sc=False),
    )
    def emb(ids_hbm, wt_hbm, out_hbm, ids_v, rows0, rows1, g0, g1, s0, s1):
        cid = lax.axis_index("c")
        sid = lax.axis_index("s")
        wid = sid * num_cores + cid
        base = wid * per_w

        pltpu.sync_copy(ids_hbm.at[pl.ds(base, per_w)], ids_v)

        def gather(i, rows, gsem):
            return pltpu.make_async_copy(wt_hbm.at[ids_v.at[i]], rows, gsem)

        def store(i, rows, ssem):
            return pltpu.make_async_copy(
                rows, out_hbm.at[:, base + i, pl.ds(0, E)], ssem)

        def step(i, rows_a, rows_b, gsem_a, gsem_b, ssem_a, ssem_b):
            gather(i, rows_a, gsem_a).wait()

            @pl.when(i > 0)
            def _():
                store(i - 1, rows_b, ssem_b).wait()

            @pl.when(i + 1 < per_w)
            def _():
                gather(i + 1, rows_b, gsem_b).start()

            store(i, rows_a, ssem_a).start()

        gather(0, rows0, g0).start()

        def pair_body(k, carry):
            step(2 * k, rows0, rows1, g0, g1, s0, s1)
            step(2 * k + 1, rows1, rows0, g1, g0, s1, s0)
            return carry

        lax.fori_loop(0, n_pair, pair_body, 0)
        store(per_w - 1, rows1, s1).wait()

    return emb


def _build_tc_kernel(B, L, E):
    def body(in_ref, pos_ref, eye_ref, out_ref):
        l = pl.program_id(0)
        x = in_ref[0][:, :E]
        xt = lax.dot_general(eye_ref[...], x, (((0,), (1,)), ((), ())),
                             preferred_element_type=jnp.float32)
        p = pos_ref[l, :]
        out_ref[0] = xt + p[:, None]

    return pl.pallas_call(
        body,
        grid=(L,),
        in_specs=[
            pl.BlockSpec((1, B, 128), lambda l: (l, 0, 0)),
            pl.BlockSpec((512, E), lambda l: (0, 0)),
            pl.BlockSpec((E, E), lambda l: (0, 0)),
        ],
        out_specs=pl.BlockSpec((1, E, B), lambda l: (l, 0, 0)),
        out_shape=jax.ShapeDtypeStruct((L, E, B), jnp.float32),
    )


def kernel(input_ids, word_table, pos_table):
    B, L = input_ids.shape
    E = word_table.shape[1]
    info = plsc.get_sparse_core_info()
    n_workers = info.num_cores * info.num_subcores

    emb = _build_sc_kernel(B, L, E, n_workers, info.num_cores)
    inter = emb(input_ids.astype(jnp.int32), word_table)
    eye = jnp.eye(E, dtype=jnp.float32)
    out_leb = _build_tc_kernel(B, L, E)(inter, pos_table, eye)
    return jnp.transpose(out_leb, (2, 0, 1))
